# baseline (speedup 1.0000x reference)
import functools

import jax
import jax.numpy as jnp
from jax import lax
from jax.experimental import pallas as pl
from jax.experimental.pallas import tpu as pltpu
from jax.experimental.pallas import tpu_sc as plsc

HIDDEN = 3072
SEQ = 4096
NUM_TOKENS = 2 * SEQ
NC = 2
NS = 16
NW = NC * NS
PER_W = NUM_TOKENS // NW
CHUNK = 8
NCHUNK = PER_W // CHUNK
NBUF = 4

_mesh = plsc.VectorSubcoreMesh(core_axis_name="c", subcore_axis_name="s")


@functools.partial(
    pl.kernel,
    mesh=_mesh,
    out_type=jax.ShapeDtypeStruct((2, SEQ, HIDDEN), jnp.float32),
    scratch_types=[
        pltpu.VMEM((PER_W,), jnp.int32),
        pltpu.VMEM((NBUF, CHUNK, HIDDEN), jnp.float32),
        pltpu.SemaphoreType.DMA,
        pltpu.SemaphoreType.DMA,
        pltpu.SemaphoreType.DMA,
        pltpu.SemaphoreType.DMA,
        pltpu.SemaphoreType.DMA,
        pltpu.SemaphoreType.DMA,
        pltpu.SemaphoreType.DMA,
        pltpu.SemaphoreType.DMA,
    ],
)
def _embed_lookup(
    table_hbm, ids_hbm, out_hbm, idx_v, rows_v,
    si0, si1, si2, si3, so0, so1, so2, so3,
):
    in_sem = (si0, si1, si2, si3)
    out_sem = (so0, so1, so2, so3)
    wid = lax.axis_index("s") * NC + lax.axis_index("c")
    tiles_per_row = SEQ // PER_W
    brow = wid // tiles_per_row
    bcol = (wid % tiles_per_row) * PER_W
    pltpu.sync_copy(ids_hbm.at[brow, pl.ds(bcol, PER_W)], idx_v)

    def gather(j, b):
        return pltpu.async_copy(
            table_hbm.at[idx_v.at[pl.ds(j * CHUNK, CHUNK)]], rows_v.at[b], in_sem[b]
        )

    def put(j, b):
        return pltpu.async_copy(
            rows_v.at[b], out_hbm.at[brow, pl.ds(bcol + j * CHUNK, CHUNK)], out_sem[b]
        )

    gcp = [gather(b, b) for b in range(NBUF)]
    pcp = [None] * NBUF
    for j in range(NCHUNK):
        b = j % NBUF
        gcp[b].wait()
        pcp[b] = put(j, b)
        if j + NBUF < NCHUNK:
            pcp[b].wait()
            gcp[b] = gather(j + NBUF, b)
    for b in range(NBUF):
        pcp[b].wait()


def kernel(embed_weight, input_ids):
    return _embed_lookup(embed_weight, input_ids.astype(jnp.int32))
